# baseline (speedup 1.0000x reference)
import functools

import jax
import jax.numpy as jnp
from jax import lax
from jax.experimental import pallas as pl
from jax.experimental.pallas import tpu as pltpu
from jax.experimental.pallas import tpu_sc as plsc

N = 10000
E = 320000
D = 128

NC = 2
NS = 16
NW = NC * NS
N_PAD = 10240
K = 64
EPW = 10240
NCHUNK = EPW // K
E_PAD = NW * EPW
ROWS_PT = N_PAD // NS

CH_F = 320
NCHUNK_E = NS * CH_F
QBUF = CH_F // 4
CPAD = NCHUNK_E + 2 * K

_mesh = plsc.VectorSubcoreMesh(
    core_axis_name="c", subcore_axis_name="s", num_cores=NC, num_subcores=NS
)


def _fill_idx(idxv, base):
    for q in range(K // 16):
        idxv[0, pl.ds(q * 16, 16)] = lax.iota(jnp.int32, 16) + (base + q * 16)


def _edges_body(msc_hbm, src2_hbm, dst2_hbm, out_hbm, srcv, dstv, rows0, rows1,
                idxz, shared, semA, semB):
    c = lax.axis_index("c")
    s = lax.axis_index("s")

    fast = c == 0

    @pl.when(fast)
    def _body():
        def _zrow(r, _):
            for q in range(D // 16):
                rows0[r, pl.ds(q * 16, 16)] = jnp.zeros((16,), jnp.float32)
            return 0

        lax.fori_loop(0, K, _zrow, 0)

        def _zscat(t, _):
            zbase = s * ROWS_PT + t * K
            for q in range(K // 16):
                idxz[0, pl.ds(q * 16, 16)] = lax.iota(jnp.int32, 16) + (zbase + q * 16)
            pltpu.sync_copy(rows0, shared.at[idxz.at[0]])
            return 0

        lax.fori_loop(0, ROWS_PT // K, _zscat, 0)
        plsc.subcore_barrier()

        nq = CH_F // 4
        pairs = nq // 2
        base = s * CH_F
        for h in range(4):
            qbase = base + h * nq
            pltpu.sync_copy(src2_hbm.at[pl.ds(qbase, QBUF)], srcv)
            pltpu.sync_copy(dst2_hbm.at[pl.ds(qbase, QBUF)], dstv)
            pltpu.async_copy(msc_hbm.at[srcv.at[0]], rows0, semA)

            def _pair(i, _):
                j0 = 2 * i
                j1 = j0 + 1
                pltpu.async_copy(msc_hbm.at[srcv.at[j1]], rows1, semB)
                pltpu.make_async_copy(msc_hbm.at[srcv.at[j0]], rows0, semA).wait()
                pltpu.sync_copy(rows0, shared.at[dstv.at[j0]], add=True)

                @pl.when(i < pairs - 1)
                def _():
                    pltpu.async_copy(msc_hbm.at[srcv.at[j0 + 2]], rows0, semA)

                pltpu.make_async_copy(msc_hbm.at[srcv.at[j1]], rows1, semB).wait()
                pltpu.sync_copy(rows1, shared.at[dstv.at[j1]], add=True)
                return 0

            lax.fori_loop(0, pairs, _pair, 0)

    plsc.subcore_barrier()

    @pl.when(fast)
    def _export():
        def _exp(t, _):
            pltpu.sync_copy(shared.at[pl.ds(s * ROWS_PT + t * K, K)], rows0)
            pltpu.sync_copy(rows0, out_hbm.at[pl.ds(s * ROWS_PT + t * K, K)])
            return 0

        lax.fori_loop(0, ROWS_PT // K, _exp, 0)


_sc_edges = pl.kernel(
    _edges_body,
    out_type=jax.ShapeDtypeStruct((N_PAD, D), jnp.float32),
    mesh=_mesh,
    scratch_types=[
        pltpu.VMEM((QBUF, K), jnp.int32),
        pltpu.VMEM((QBUF, K), jnp.int32),
        pltpu.VMEM((K, D), jnp.float32),
        pltpu.VMEM((K, D), jnp.float32),
        pltpu.VMEM((1, K), jnp.int32),
        pltpu.VMEM_SHARED((N_PAD, D), jnp.float32),
        pltpu.SemaphoreType.DMA,
        pltpu.SemaphoreType.DMA,
    ],
)


def _degree_body(dst3_hbm, out_hbm, dstv, valbuf, idxz, shared):
    c = lax.axis_index("c")
    s = lax.axis_index("s")
    wid = s * NC + c

    def _zrow(r, _):
        for q in range(D // 16):
            valbuf[r, pl.ds(q * 16, 16)] = jnp.zeros((16,), jnp.float32)
        return 0

    lax.fori_loop(0, K, _zrow, 0)
    for t in range(ROWS_PT // K):
        _fill_idx(idxz, s * ROWS_PT + t * K)
        pltpu.sync_copy(valbuf, shared.at[idxz.at[0]])

    def _orow(r, _):
        for q in range(D // 16):
            valbuf[r, pl.ds(q * 16, 16)] = jnp.ones((16,), jnp.float32)
        return 0

    lax.fori_loop(0, K, _orow, 0)
    pltpu.sync_copy(dst3_hbm.at[wid], dstv)
    plsc.subcore_barrier()

    def _chunk(j, _):
        pltpu.sync_copy(valbuf, shared.at[dstv.at[j]], add=True)
        return 0

    lax.fori_loop(0, NCHUNK, _chunk, 0)
    plsc.subcore_barrier()
    for t in range(ROWS_PT // K):
        pltpu.sync_copy(shared.at[pl.ds(s * ROWS_PT + t * K, K)], valbuf)
        pltpu.sync_copy(
            valbuf, out_hbm.at[pl.ds(c * N_PAD + s * ROWS_PT + t * K, K)]
        )


_sc_degree = pl.kernel(
    _degree_body,
    out_type=jax.ShapeDtypeStruct((2 * N_PAD, D), jnp.float32),
    mesh=_mesh,
    scratch_types=[
        pltpu.VMEM((NCHUNK, K), jnp.int32),
        pltpu.VMEM((K, D), jnp.float32),
        pltpu.VMEM((1, K), jnp.int32),
        pltpu.VMEM_SHARED((N_PAD, D), jnp.float32),
    ],
)


BLK = 1024
GRID = N_PAD // BLK

_row_spec = pl.BlockSpec((BLK, D), lambda i: (i, 0))
_full_spec = pl.BlockSpec((D, D), lambda i: (0, 0))
_b_spec = pl.BlockSpec((1, D), lambda i: (0, 0))
_acc_spec = pl.BlockSpec((2, BLK, D), lambda i: (0, i, 0))


def _tc_in_body(x_ref, win_ref, bin_ref, w1_ref, deg_ref, h0_ref, dinv_ref, msc_ref):
    xb = x_ref[...]
    h0 = jnp.maximum(jnp.dot(xb, win_ref[...], preferred_element_type=jnp.float32)
                     + bin_ref[...], 0.0)
    deg = deg_ref[0] + deg_ref[1] + 1.0
    dinv_b = jax.lax.rsqrt(deg)
    h0_ref[...] = h0
    dinv_ref[...] = dinv_b
    msc_ref[...] = dinv_b * jnp.dot(h0, w1_ref[...], preferred_element_type=jnp.float32)


_tc_input = pl.pallas_call(
    _tc_in_body,
    grid=(GRID,),
    in_specs=[_row_spec, _full_spec, _b_spec, _full_spec, _acc_spec],
    out_specs=[_row_spec, _row_spec, _row_spec],
    out_shape=[
        jax.ShapeDtypeStruct((N_PAD, D), jnp.float32),
        jax.ShapeDtypeStruct((N_PAD, D), jnp.float32),
        jax.ShapeDtypeStruct((N_PAD, D), jnp.float32),
    ],
)


def _tc_layer_body(h_ref, msc_ref, acc_ref, dinv_ref, b_ref, wn_ref,
                   hn_ref, mscn_ref):
    dinv_b = dinv_ref[...]
    agg = acc_ref[...] + msc_ref[...]
    conv = dinv_b * agg + b_ref[...]
    h_new = h_ref[...] + jnp.maximum(conv, 0.0)
    hn_ref[...] = h_new
    mscn_ref[...] = dinv_b * jnp.dot(h_new, wn_ref[...],
                                     preferred_element_type=jnp.float32)


_tc_layer = pl.pallas_call(
    _tc_layer_body,
    grid=(GRID,),
    in_specs=[_row_spec, _row_spec, _row_spec, _row_spec, _b_spec, _full_spec],
    out_specs=[_row_spec, _row_spec],
    out_shape=[
        jax.ShapeDtypeStruct((N_PAD, D), jnp.float32),
        jax.ShapeDtypeStruct((N_PAD, D), jnp.float32),
    ],
)


def _tc_out_body(h_ref, msc_ref, acc_ref, dinv_ref, b_ref, wo_ref, bo_ref, o_ref):
    dinv_b = dinv_ref[...]
    agg = acc_ref[...] + msc_ref[...]
    conv = dinv_b * agg + b_ref[...]
    h_new = h_ref[...] + jnp.maximum(conv, 0.0)
    o_ref[...] = jnp.dot(h_new, wo_ref[...],
                         preferred_element_type=jnp.float32) + bo_ref[...]


_tc_out = pl.pallas_call(
    _tc_out_body,
    grid=(GRID,),
    in_specs=[_row_spec, _row_spec, _row_spec, _row_spec, _b_spec, _full_spec,
              _b_spec],
    out_specs=_row_spec,
    out_shape=jax.ShapeDtypeStruct((N_PAD, D), jnp.float32),
)


def kernel(x, edge_index, W_in, b_in, W1, b1, W2, b2, W3, b3, W_out, b_out):
    src = edge_index[0]
    dst = edge_index[1]
    npad = CPAD * K - E
    pad_src = jnp.zeros((npad,), jnp.int32)
    pad_dst = N + (jnp.arange(npad, dtype=jnp.int32) % (N_PAD - N))
    src2 = jnp.concatenate([src, pad_src]).reshape(CPAD, K)
    dst2 = jnp.concatenate([dst, pad_dst]).reshape(CPAD, K)
    dst3 = dst2[:NCHUNK_E].reshape(NW, NCHUNK, K)
    x_pad = jnp.zeros((N_PAD, D), jnp.float32).at[:N].set(x.astype(jnp.float32))

    deg_flat = _sc_degree(dst3)
    deg2 = deg_flat.reshape(2, N_PAD, D)

    h0, dinv_b, msc1 = _tc_input(x_pad, W_in, b_in.reshape(1, D), W1, deg2)

    acc1 = _sc_edges(msc1, src2, dst2)
    h1, msc2 = _tc_layer(h0, msc1, acc1, dinv_b, b1.reshape(1, D), W2)

    acc2 = _sc_edges(msc2, src2, dst2)
    h2, msc3 = _tc_layer(h1, msc2, acc2, dinv_b, b2.reshape(1, D), W3)

    acc3 = _sc_edges(msc3, src2, dst2)
    out = _tc_out(h2, msc3, acc3, dinv_b, b3.reshape(1, D), W_out,
                  b_out.reshape(1, D))
    return out[:N]


# speedup vs baseline: 1.3029x; 1.3029x over previous
import functools

import jax
import jax.numpy as jnp
from jax import lax
from jax.experimental import pallas as pl
from jax.experimental.pallas import tpu as pltpu
from jax.experimental.pallas import tpu_sc as plsc

N = 10000
E = 320000
D = 128

NC = 2
NS = 16
NW = NC * NS
N_PAD = 10240
K = 64
EPW = 10240
NCHUNK = EPW // K
E_PAD = NW * EPW
ROWS_PT = N_PAD // NS

CH_F = 288
CH_S = 32
FAST_TOT = NS * CH_F
NCHUNK_E = FAST_TOT + NS * CH_S
QBUF = max(CH_F, CH_S) // 4
CPAD = NCHUNK_E + 2 * K

_mesh = plsc.VectorSubcoreMesh(
    core_axis_name="c", subcore_axis_name="s", num_cores=NC, num_subcores=NS
)


def _edges_body(msc_hbm, src2_hbm, dst2_hbm, zeros_hbm, idz3_hbm, out_hbm,
                srcv, dstv, rows0, rows1, idxz, shared, semA, semB):
    c = lax.axis_index("c")
    s = lax.axis_index("s")

    pltpu.sync_copy(zeros_hbm, rows0)
    pltpu.sync_copy(idz3_hbm.at[s], idxz)

    def _zscat(t, _):
        pltpu.sync_copy(rows0, shared.at[idxz.at[t]])
        return 0

    lax.fori_loop(0, ROWS_PT // K, _zscat, 0)
    plsc.subcore_barrier()

    fast = c == 0
    nq = jnp.where(fast, CH_F // 4, CH_S // 4)
    base = jnp.where(fast, s * CH_F, FAST_TOT + s * CH_S)
    pairs = nq // 2
    for h in range(4):
        qbase = base + h * nq
        pltpu.sync_copy(src2_hbm.at[pl.ds(qbase, QBUF)], srcv)
        pltpu.sync_copy(dst2_hbm.at[pl.ds(qbase, QBUF)], dstv)
        pltpu.async_copy(msc_hbm.at[srcv.at[0]], rows0, semA)

        def _pair(i, _):
            j0 = 2 * i
            j1 = j0 + 1
            pltpu.async_copy(msc_hbm.at[srcv.at[j1]], rows1, semB)
            pltpu.make_async_copy(msc_hbm.at[srcv.at[j0]], rows0, semA).wait()
            pltpu.sync_copy(rows0, shared.at[dstv.at[j0]], add=True)

            @pl.when(i < pairs - 1)
            def _():
                pltpu.async_copy(msc_hbm.at[srcv.at[j0 + 2]], rows0, semA)

            pltpu.make_async_copy(msc_hbm.at[srcv.at[j1]], rows1, semB).wait()
            pltpu.sync_copy(rows1, shared.at[dstv.at[j1]], add=True)
            return 0

        lax.fori_loop(0, pairs, _pair, 0)
    plsc.subcore_barrier()

    def _exp(t, _):
        pltpu.sync_copy(shared.at[pl.ds(s * ROWS_PT + t * K, K)], rows0)
        pltpu.sync_copy(rows0, out_hbm.at[pl.ds(c * N_PAD + s * ROWS_PT + t * K, K)])
        return 0

    lax.fori_loop(0, ROWS_PT // K, _exp, 0)


_sc_edges = pl.kernel(
    _edges_body,
    out_type=jax.ShapeDtypeStruct((2 * N_PAD, D), jnp.float32),
    mesh=_mesh,
    scratch_types=[
        pltpu.VMEM((QBUF, K), jnp.int32),
        pltpu.VMEM((QBUF, K), jnp.int32),
        pltpu.VMEM((K, D), jnp.float32),
        pltpu.VMEM((K, D), jnp.float32),
        pltpu.VMEM((16, K), jnp.int32),
        pltpu.VMEM_SHARED((N_PAD, D), jnp.float32),
        pltpu.SemaphoreType.DMA,
        pltpu.SemaphoreType.DMA,
    ],
)


def _degree_body(dst3_hbm, zeros_hbm, ones_hbm, idz3_hbm, out_hbm, dstv,
                 valbuf, idxz, shared):
    c = lax.axis_index("c")
    s = lax.axis_index("s")
    wid = s * NC + c

    pltpu.sync_copy(zeros_hbm, valbuf)
    pltpu.sync_copy(idz3_hbm.at[s], idxz)

    def _zscat(t, _):
        pltpu.sync_copy(valbuf, shared.at[idxz.at[t]])
        return 0

    lax.fori_loop(0, ROWS_PT // K, _zscat, 0)
    pltpu.sync_copy(ones_hbm, valbuf)
    pltpu.sync_copy(dst3_hbm.at[wid], dstv)
    plsc.subcore_barrier()

    def _chunk(j, _):
        pltpu.sync_copy(valbuf, shared.at[dstv.at[j]], add=True)
        return 0

    lax.fori_loop(0, NCHUNK, _chunk, 0)
    plsc.subcore_barrier()
    for t in range(ROWS_PT // K):
        pltpu.sync_copy(shared.at[pl.ds(s * ROWS_PT + t * K, K)], valbuf)
        pltpu.sync_copy(
            valbuf, out_hbm.at[pl.ds(c * N_PAD + s * ROWS_PT + t * K, K)]
        )


_sc_degree = pl.kernel(
    _degree_body,
    out_type=jax.ShapeDtypeStruct((2 * N_PAD, D), jnp.float32),
    mesh=_mesh,
    scratch_types=[
        pltpu.VMEM((NCHUNK, K), jnp.int32),
        pltpu.VMEM((K, D), jnp.float32),
        pltpu.VMEM((16, K), jnp.int32),
        pltpu.VMEM_SHARED((N_PAD, D), jnp.float32),
    ],
)


BLK = 1024
GRID = N_PAD // BLK

_row_spec = pl.BlockSpec((BLK, D), lambda i: (i, 0))
_full_spec = pl.BlockSpec((D, D), lambda i: (0, 0))
_b_spec = pl.BlockSpec((1, D), lambda i: (0, 0))
_acc_spec = pl.BlockSpec((2, BLK, D), lambda i: (0, i, 0))


def _tc_in_body(x_ref, win_ref, bin_ref, w1_ref, deg_ref, h0_ref, dinv_ref, msc_ref):
    xb = x_ref[...]
    h0 = jnp.maximum(jnp.dot(xb, win_ref[...], preferred_element_type=jnp.float32)
                     + bin_ref[...], 0.0)
    deg = deg_ref[0] + deg_ref[1] + 1.0
    dinv_b = jax.lax.rsqrt(deg)
    h0_ref[...] = h0
    dinv_ref[...] = dinv_b
    msc_ref[...] = dinv_b * jnp.dot(h0, w1_ref[...], preferred_element_type=jnp.float32)


_tc_input = pl.pallas_call(
    _tc_in_body,
    grid=(GRID,),
    in_specs=[_row_spec, _full_spec, _b_spec, _full_spec, _acc_spec],
    out_specs=[_row_spec, _row_spec, _row_spec],
    out_shape=[
        jax.ShapeDtypeStruct((N_PAD, D), jnp.float32),
        jax.ShapeDtypeStruct((N_PAD, D), jnp.float32),
        jax.ShapeDtypeStruct((N_PAD, D), jnp.float32),
    ],
)


def _tc_layer_body(h_ref, msc_ref, acc_ref, dinv_ref, b_ref, wn_ref,
                   hn_ref, mscn_ref):
    dinv_b = dinv_ref[...]
    agg = acc_ref[0] + acc_ref[1] + msc_ref[...]
    conv = dinv_b * agg + b_ref[...]
    h_new = h_ref[...] + jnp.maximum(conv, 0.0)
    hn_ref[...] = h_new
    mscn_ref[...] = dinv_b * jnp.dot(h_new, wn_ref[...],
                                     preferred_element_type=jnp.float32)


_tc_layer = pl.pallas_call(
    _tc_layer_body,
    grid=(GRID,),
    in_specs=[_row_spec, _row_spec, _acc_spec, _row_spec, _b_spec, _full_spec],
    out_specs=[_row_spec, _row_spec],
    out_shape=[
        jax.ShapeDtypeStruct((N_PAD, D), jnp.float32),
        jax.ShapeDtypeStruct((N_PAD, D), jnp.float32),
    ],
)


def _tc_out_body(h_ref, msc_ref, acc_ref, dinv_ref, b_ref, wo_ref, bo_ref, o_ref):
    dinv_b = dinv_ref[...]
    agg = acc_ref[0] + acc_ref[1] + msc_ref[...]
    conv = dinv_b * agg + b_ref[...]
    h_new = h_ref[...] + jnp.maximum(conv, 0.0)
    o_ref[...] = jnp.dot(h_new, wo_ref[...],
                         preferred_element_type=jnp.float32) + bo_ref[...]


_tc_out = pl.pallas_call(
    _tc_out_body,
    grid=(GRID,),
    in_specs=[_row_spec, _row_spec, _acc_spec, _row_spec, _b_spec, _full_spec,
              _b_spec],
    out_specs=_row_spec,
    out_shape=jax.ShapeDtypeStruct((N_PAD, D), jnp.float32),
)


def kernel(x, edge_index, W_in, b_in, W1, b1, W2, b2, W3, b3, W_out, b_out):
    src = edge_index[0]
    dst = edge_index[1]
    npad = CPAD * K - E
    pad_src = jnp.zeros((npad,), jnp.int32)
    pad_dst = N + (jnp.arange(npad, dtype=jnp.int32) % (N_PAD - N))
    src2 = jnp.concatenate([src, pad_src]).reshape(CPAD, K)
    dst2 = jnp.concatenate([dst, pad_dst]).reshape(CPAD, K)
    dst3 = dst2[:NCHUNK_E].reshape(NW, NCHUNK, K)
    x_pad = jnp.zeros((N_PAD, D), jnp.float32).at[:N].set(x.astype(jnp.float32))
    zeros_kd = jnp.zeros((K, D), jnp.float32)
    ones_kd = jnp.ones((K, D), jnp.float32)
    idz3 = jnp.pad(
        jnp.arange(N_PAD, dtype=jnp.int32).reshape(NS, ROWS_PT // K, K),
        ((0, 0), (0, 16 - ROWS_PT // K), (0, 0)),
    )

    deg_flat = _sc_degree(dst3, zeros_kd, ones_kd, idz3)
    deg2 = deg_flat.reshape(2, N_PAD, D)

    h0, dinv_b, msc1 = _tc_input(x_pad, W_in, b_in.reshape(1, D), W1, deg2)

    acc1 = _sc_edges(msc1, src2, dst2, zeros_kd, idz3).reshape(2, N_PAD, D)
    h1, msc2 = _tc_layer(h0, msc1, acc1, dinv_b, b1.reshape(1, D), W2)

    acc2 = _sc_edges(msc2, src2, dst2, zeros_kd, idz3).reshape(2, N_PAD, D)
    h2, msc3 = _tc_layer(h1, msc2, acc2, dinv_b, b2.reshape(1, D), W3)

    acc3 = _sc_edges(msc3, src2, dst2, zeros_kd, idz3).reshape(2, N_PAD, D)
    out = _tc_out(h2, msc3, acc3, dinv_b, b3.reshape(1, D), W_out,
                  b_out.reshape(1, D))
    return out[:N]
